# initial kernel scaffold (unmeasured)
import jax
import jax.numpy as jnp
from jax import lax
from jax.experimental import pallas as pl
from jax.experimental.pallas import tpu as pltpu


def kernel(
    x,
):
    def body(*refs):
        pass

    out_shape = jax.ShapeDtypeStruct(..., jnp.float32)
    return pl.pallas_call(body, out_shape=out_shape)(...)


# baseline (device time: 19379 ns/iter reference)
import jax
import jax.numpy as jnp
from jax import lax
from jax.experimental import pallas as pl
from jax.experimental.pallas import tpu as pltpu

N_DEV = 4


def kernel(x):
    _, m, n = x.shape
    chunk = n // N_DEV

    def body(x_ref, out_ref, sbuf, rbuf, send_sems, recv_sems):
        my_x = lax.axis_index("x")
        my_y = lax.axis_index("y")
        my_z = lax.axis_index("z")
        left = lax.rem(my_z + N_DEV - 1, N_DEV)
        right = lax.rem(my_z + 1, N_DEV)

        barrier_sem = pltpu.get_barrier_semaphore()
        for nbr in (left, right):
            pl.semaphore_signal(
                barrier_sem,
                inc=1,
                device_id=(my_x, my_y, nbr),
                device_id_type=pl.DeviceIdType.MESH,
            )
        pl.semaphore_wait(barrier_sem, 2)

        for h in range(N_DEV - 1):
            c = lax.rem(my_z + (N_DEV - 1 - h), N_DEV)
            contrib = x_ref[0, :, pl.ds(c * chunk, chunk)]
            if h == 0:
                sbuf[h, :, :] = contrib
            else:
                sbuf[h, :, :] = rbuf[h - 1, :, :] + contrib
            rdma = pltpu.make_async_remote_copy(
                src_ref=sbuf.at[h],
                dst_ref=rbuf.at[h],
                send_sem=send_sems.at[h],
                recv_sem=recv_sems.at[h],
                device_id=(my_x, my_y, right),
                device_id_type=pl.DeviceIdType.MESH,
            )
            rdma.start()
            rdma.wait()

        out_ref[:, :] = rbuf[N_DEV - 2, :, :] + x_ref[0, :, pl.ds(my_z * chunk, chunk)]

    return pl.pallas_call(
        body,
        out_shape=jax.ShapeDtypeStruct((m, chunk), jnp.float32),
        in_specs=[pl.BlockSpec(memory_space=pltpu.VMEM)],
        out_specs=pl.BlockSpec(memory_space=pltpu.VMEM),
        scratch_shapes=[
            pltpu.VMEM((N_DEV - 1, m, chunk), jnp.float32),
            pltpu.VMEM((N_DEV - 1, m, chunk), jnp.float32),
            pltpu.SemaphoreType.DMA((N_DEV - 1,)),
            pltpu.SemaphoreType.DMA((N_DEV - 1,)),
        ],
        compiler_params=pltpu.CompilerParams(collective_id=0),
    )(x)


# device time: 15372 ns/iter; 1.2607x vs baseline; 1.2607x over previous
import jax
import jax.numpy as jnp
from jax import lax
from jax.experimental import pallas as pl
from jax.experimental.pallas import tpu as pltpu

N_DEV = 4


def kernel(x):
    _, m, n = x.shape
    chunk = n // N_DEV
    half = chunk // 2

    def body(x_ref, out_ref, t_l, t_r, d_l, d_r, f_l, f_r, fwd_r, fwd_l, sems):
        my_x = lax.axis_index("x")
        my_y = lax.axis_index("y")
        my_z = lax.axis_index("z")
        left = lax.rem(my_z + N_DEV - 1, N_DEV)
        right = lax.rem(my_z + 1, N_DEV)
        opp = lax.rem(my_z + 2, N_DEV)

        def xs(c, h):
            return x_ref[0, :, pl.ds(c * chunk + h * half, half)]

        def xr(c, h):
            return x_ref.at[0, :, pl.ds(c * chunk + h * half, half)]

        barrier_sem = pltpu.get_barrier_semaphore()
        for nbr in (left, right):
            pl.semaphore_signal(
                barrier_sem,
                inc=1,
                device_id=(my_x, my_y, nbr),
                device_id_type=pl.DeviceIdType.MESH,
            )
        pl.semaphore_wait(barrier_sem, 2)

        def copy(src, dst, sem_idx, dev_z):
            return pltpu.make_async_remote_copy(
                src_ref=src,
                dst_ref=dst,
                send_sem=sems.at[sem_idx],
                recv_sem=sems.at[sem_idx + 6],
                device_id=(my_x, my_y, dev_z),
                device_id_type=pl.DeviceIdType.MESH,
            )

        r_t_r = copy(xr(opp, 0), t_l, 0, right)
        r_t_l = copy(xr(opp, 1), t_r, 1, left)
        r_t_r.start()
        r_t_l.start()
        r_d_r = copy(xr(right, 1), d_l, 2, right)
        r_d_l = copy(xr(left, 0), d_r, 3, left)
        r_d_r.start()
        r_d_l.start()

        r_t_r.wait()
        fwd_r[:, :] = xs(right, 0) + t_l[:, :]
        r_f_r = copy(fwd_r, f_l, 4, right)
        r_f_r.start()

        r_t_l.wait()
        fwd_l[:, :] = xs(left, 1) + t_r[:, :]
        r_f_l = copy(fwd_l, f_r, 5, left)
        r_f_l.start()

        r_d_r.wait()
        r_d_l.wait()
        r_f_r.wait()
        r_f_l.wait()
        out_ref[:, pl.ds(0, half)] = xs(my_z, 0) + d_r[:, :] + f_l[:, :]
        out_ref[:, pl.ds(half, half)] = xs(my_z, 1) + d_l[:, :] + f_r[:, :]

    return pl.pallas_call(
        body,
        out_shape=jax.ShapeDtypeStruct((m, chunk), jnp.float32),
        in_specs=[pl.BlockSpec(memory_space=pltpu.VMEM)],
        out_specs=pl.BlockSpec(memory_space=pltpu.VMEM),
        scratch_shapes=[
            pltpu.VMEM((m, half), jnp.float32),
            pltpu.VMEM((m, half), jnp.float32),
            pltpu.VMEM((m, half), jnp.float32),
            pltpu.VMEM((m, half), jnp.float32),
            pltpu.VMEM((m, half), jnp.float32),
            pltpu.VMEM((m, half), jnp.float32),
            pltpu.VMEM((m, half), jnp.float32),
            pltpu.VMEM((m, half), jnp.float32),
            pltpu.SemaphoreType.DMA((12,)),
        ],
        compiler_params=pltpu.CompilerParams(collective_id=0),
    )(x)
